# baseline (speedup 1.0000x reference)
import functools
import jax
import jax.numpy as jnp
from jax import lax
from jax.experimental import pallas as pl
from jax.experimental.pallas import tpu as pltpu, tpu_sc as plsc

N_ROWS_BLK = 1000
E_BLK = 1024
_TC_PAR = pltpu.CompilerParams(dimension_semantics=("parallel",))


def _k1_body(x_ref, w_ref, b_ref, dinv_ref, o_ref):
    m = jnp.dot(x_ref[...], w_ref[...], preferred_element_type=jnp.float32)
    o_ref[...] = (m + b_ref[...]) * dinv_ref[...]


def _tc_linear_scale(x, W, b, dinv2):
    n, d = x.shape
    grid = (n // N_ROWS_BLK,)
    return pl.pallas_call(
        _k1_body,
        grid=grid,
        compiler_params=_TC_PAR,
        in_specs=[
            pl.BlockSpec((N_ROWS_BLK, d), lambda i: (i, 0)),
            pl.BlockSpec((d, d), lambda i: (0, 0)),
            pl.BlockSpec((1, d), lambda i: (0, 0)),
            pl.BlockSpec((N_ROWS_BLK, 1), lambda i: (i, 0)),
        ],
        out_specs=pl.BlockSpec((N_ROWS_BLK, d), lambda i: (i, 0)),
        out_shape=jax.ShapeDtypeStruct((n, d), jnp.float32),
    )(x, W, b, dinv2)


def _k2_body(s_ref, m_ref, dinv_ref, w_ref, b_ref, h_ref, o_ref):
    ssum = s_ref[0] + s_ref[1] + m_ref[...]
    h1 = jnp.maximum(ssum * dinv_ref[...], 0.0)
    h_ref[...] = h1
    m2 = jnp.dot(h1, w_ref[...], preferred_element_type=jnp.float32)
    o_ref[...] = (m2 + b_ref[...]) * dinv_ref[...]


def _tc_combine_next(S, m1s, dinv2, W2, b2):
    n, d = m1s.shape
    grid = (n // N_ROWS_BLK,)
    return pl.pallas_call(
        _k2_body,
        grid=grid,
        compiler_params=_TC_PAR,
        in_specs=[
            pl.BlockSpec((2, N_ROWS_BLK, d), lambda i: (0, i, 0)),
            pl.BlockSpec((N_ROWS_BLK, d), lambda i: (i, 0)),
            pl.BlockSpec((N_ROWS_BLK, 1), lambda i: (i, 0)),
            pl.BlockSpec((d, d), lambda i: (0, 0)),
            pl.BlockSpec((1, d), lambda i: (0, 0)),
        ],
        out_specs=[
            pl.BlockSpec((N_ROWS_BLK, d), lambda i: (i, 0)),
            pl.BlockSpec((N_ROWS_BLK, d), lambda i: (i, 0)),
        ],
        out_shape=[
            jax.ShapeDtypeStruct((n, d), jnp.float32),
            jax.ShapeDtypeStruct((n, d), jnp.float32),
        ],
    )(S, m1s, dinv2, W2, b2[None, :])


def _k3_body(s_ref, m_ref, dinv_ref, h1_ref, o_ref):
    ssum = s_ref[0] + s_ref[1] + m_ref[...]
    o_ref[...] = jnp.maximum(ssum * dinv_ref[...], 0.0) + h1_ref[...]


def _tc_final_combine(S, m2s, dinv2, h1):
    n, d = m2s.shape
    grid = (n // N_ROWS_BLK,)
    return pl.pallas_call(
        _k3_body,
        grid=grid,
        compiler_params=_TC_PAR,
        in_specs=[
            pl.BlockSpec((2, N_ROWS_BLK, d), lambda i: (0, i, 0)),
            pl.BlockSpec((N_ROWS_BLK, d), lambda i: (i, 0)),
            pl.BlockSpec((N_ROWS_BLK, 1), lambda i: (i, 0)),
            pl.BlockSpec((N_ROWS_BLK, d), lambda i: (i, 0)),
        ],
        out_specs=pl.BlockSpec((N_ROWS_BLK, d), lambda i: (i, 0)),
        out_shape=jax.ShapeDtypeStruct((n, d), jnp.float32),
    )(S, m2s, dinv2, h1)


def _head_body(hr_ref, hc_ref, zl_ref, rel_ref, w1_ref, w1l_ref, b1_ref,
               w2_ref, b2_ref, lanes_ref, wr2d_ref, brc_ref, ones_ref, o_ref):
    z = (hr_ref[...] * hc_ref[...]).astype(jnp.bfloat16)
    zl = zl_ref[...]
    a = jnp.dot(z, w1_ref[...], preferred_element_type=jnp.float32)
    a = a + zl * w1l_ref[...] + b1_ref[...]
    a = jnp.maximum(a, 0.0).astype(jnp.bfloat16)
    a = jnp.dot(a, w2_ref[...], preferred_element_type=jnp.float32) + b2_ref[...]
    a = jnp.maximum(a, 0.0)
    onehot = jnp.where(rel_ref[...] == lanes_ref[...], 1.0, 0.0)
    w = jnp.dot(onehot, wr2d_ref[...], preferred_element_type=jnp.float32)
    s1 = jnp.dot(a * w, ones_ref[...], preferred_element_type=jnp.float32)
    s2 = jnp.dot(onehot, brc_ref[...], preferred_element_type=jnp.float32)
    o_ref[...] = (s1 + s2)[:, 0]


def _tc_head(hr, hc, zlast2, rel2, Wl1, bl1, Wl2, bl2, Wr, br):
    e, d = hr.shape
    w1m = Wl1[:d].astype(jnp.bfloat16)
    w1l = Wl1[d:d + 1]
    lanes = jnp.arange(4, dtype=jnp.float32)[None, :]
    wr2d = Wr[:, :, 0]
    brc = br
    ones64 = jnp.ones((64, 1), jnp.float32)
    grid = (e // E_BLK,)
    return pl.pallas_call(
        _head_body,
        grid=grid,
        compiler_params=_TC_PAR,
        in_specs=[
            pl.BlockSpec((E_BLK, d), lambda i: (i, 0)),
            pl.BlockSpec((E_BLK, d), lambda i: (i, 0)),
            pl.BlockSpec((E_BLK, 1), lambda i: (i, 0)),
            pl.BlockSpec((E_BLK, 1), lambda i: (i, 0)),
            pl.BlockSpec((d, 256), lambda i: (0, 0)),
            pl.BlockSpec((1, 256), lambda i: (0, 0)),
            pl.BlockSpec((1, 256), lambda i: (0, 0)),
            pl.BlockSpec((256, 64), lambda i: (0, 0)),
            pl.BlockSpec((1, 64), lambda i: (0, 0)),
            pl.BlockSpec((1, 4), lambda i: (0, 0)),
            pl.BlockSpec((4, 64), lambda i: (0, 0)),
            pl.BlockSpec((4, 1), lambda i: (0, 0)),
            pl.BlockSpec((64, 1), lambda i: (0, 0)),
        ],
        out_specs=pl.BlockSpec((E_BLK,), lambda i: (i,)),
        out_shape=jax.ShapeDtypeStruct((e,), jnp.float32),
    )(hr, hc, zlast2, rel2, w1m, w1l, bl1[None, :], Wl2.astype(jnp.bfloat16),
      bl2[None, :], lanes, wr2d, brc, ones64)


_SC_CORES = 2
_SC_TILES = 16
_NW = _SC_CORES * _SC_TILES
_CHUNK = 125
_SEG = 16


def _sc_degree(tei3, zeros_n, n):
    rows_total, chunk = tei3.shape[1:]
    rows_per_tile = rows_total // _NW
    stripe = 624
    last = n - (_SC_TILES - 1) * stripe
    mesh = plsc.VectorSubcoreMesh(core_axis_name="c", subcore_axis_name="s")

    @functools.partial(
        pl.kernel,
        out_type=jax.ShapeDtypeStruct((_SC_CORES * n,), jnp.float32),
        mesh=mesh,
        scratch_types=[
            pltpu.VMEM((rows_per_tile, chunk), jnp.int32),
            pltpu.VMEM((128,), jnp.float32),
            pltpu.VMEM((640,), jnp.float32),
            pltpu.VMEM_SHARED((n,), jnp.float32),
        ],
    )
    def k(tei_hbm, z_hbm, out_hbm, dst_v, ones_v, zbuf, acc):
        del z_hbm
        cid = lax.axis_index("c")
        sid = lax.axis_index("s")
        wid = cid * _SC_TILES + sid
        dst_hbm = tei_hbm.at[1]

        @pl.loop(0, 128, step=16)
        def _(i):
            ones_v[pl.ds(i, 16)] = jnp.full((16,), 1.0, jnp.float32)

        @pl.loop(0, 640, step=16)
        def _(i):
            zbuf[pl.ds(i, 16)] = jnp.zeros((16,), jnp.float32)

        @pl.when(sid < _SC_TILES - 1)
        def _():
            pltpu.sync_copy(zbuf.at[pl.ds(0, stripe)],
                            acc.at[pl.ds(sid * stripe, stripe)])

        @pl.when(sid == _SC_TILES - 1)
        def _():
            pltpu.sync_copy(zbuf.at[pl.ds(0, last)],
                            acc.at[pl.ds(sid * stripe, last)])

        pltpu.sync_copy(dst_hbm.at[pl.ds(wid * rows_per_tile, rows_per_tile)],
                        dst_v)
        plsc.subcore_barrier()

        @pl.loop(0, rows_per_tile)
        def _(r):
            pltpu.sync_copy(ones_v.at[pl.ds(0, chunk)],
                            acc.at[dst_v.at[r]], add=True)

        plsc.subcore_barrier()

        @pl.when(sid < _SC_TILES - 1)
        def _():
            pltpu.sync_copy(acc.at[pl.ds(sid * stripe, stripe)],
                            zbuf.at[pl.ds(0, stripe)])
            pltpu.sync_copy(zbuf.at[pl.ds(0, stripe)],
                            out_hbm.at[pl.ds(cid * n + sid * stripe, stripe)])

        @pl.when(sid == _SC_TILES - 1)
        def _():
            pltpu.sync_copy(acc.at[pl.ds(sid * stripe, last)],
                            zbuf.at[pl.ds(0, last)])
            pltpu.sync_copy(zbuf.at[pl.ds(0, last)],
                            out_hbm.at[pl.ds(cid * n + sid * stripe, last)])

    return k(tei3, zeros_n)


def _sc_segsum(ms, tei3, zeros_nd):
    n, d = ms.shape
    rows_total, chunk = tei3.shape[1:]
    rows_per_tile = rows_total // _NW
    stripe = 624
    last = n - (_SC_TILES - 1) * stripe
    mesh = plsc.VectorSubcoreMesh(core_axis_name="c", subcore_axis_name="s")

    @functools.partial(
        pl.kernel,
        out_type=jax.ShapeDtypeStruct((_SC_CORES, n, d), jnp.float32),
        mesh=mesh,
        scratch_types=[
            pltpu.VMEM((_SEG, chunk), jnp.int32),
            pltpu.VMEM((_SEG, chunk), jnp.int32),
            pltpu.VMEM((chunk, d), jnp.float32),
            pltpu.VMEM((chunk, d), jnp.float32),
            pltpu.VMEM_SHARED((n, d), jnp.float32),
            pltpu.SemaphoreType.DMA,
            pltpu.SemaphoreType.DMA,
        ],
    )
    def k(ms_hbm, tei_hbm, z_hbm, out_hbm,
          src_v, dst_v, buf0, buf1, acc, sem0, sem1):
        cid = lax.axis_index("c")
        sid = lax.axis_index("s")
        wid = cid * _SC_TILES + sid
        src_hbm = tei_hbm.at[0]
        dst_hbm = tei_hbm.at[1]

        @pl.when(sid < _SC_TILES - 1)
        def _():
            pltpu.sync_copy(z_hbm.at[pl.ds(sid * stripe, stripe)],
                            acc.at[pl.ds(sid * stripe, stripe)])

        @pl.when(sid == _SC_TILES - 1)
        def _():
            pltpu.sync_copy(z_hbm.at[pl.ds(sid * stripe, last)],
                            acc.at[pl.ds(sid * stripe, last)])

        plsc.subcore_barrier()

        @pl.loop(0, rows_per_tile, step=_SEG)
        def _(seg):
            base = wid * rows_per_tile + seg
            pltpu.sync_copy(src_hbm.at[pl.ds(base, _SEG)], src_v)
            pltpu.sync_copy(dst_hbm.at[pl.ds(base, _SEG)], dst_v)
            pltpu.async_copy(ms_hbm.at[src_v.at[0]], buf0, sem0)

            @pl.loop(0, _SEG, step=2)
            def _(r):
                pltpu.async_copy(ms_hbm.at[src_v.at[r + 1]], buf1, sem1)
                pltpu.make_async_copy(ms_hbm.at[src_v.at[r]], buf0, sem0).wait()
                pltpu.sync_copy(buf0, acc.at[dst_v.at[r]], add=True)

                @pl.when(r + 2 < _SEG)
                def _():
                    pltpu.async_copy(ms_hbm.at[src_v.at[r + 2]], buf0, sem0)

                pltpu.make_async_copy(ms_hbm.at[src_v.at[r + 1]], buf1, sem1).wait()
                pltpu.sync_copy(buf1, acc.at[dst_v.at[r + 1]], add=True)

        plsc.subcore_barrier()

        @pl.when(sid < _SC_TILES - 1)
        def _():
            pltpu.sync_copy(acc.at[pl.ds(sid * stripe, stripe)],
                            out_hbm.at[cid, pl.ds(sid * stripe, stripe)])

        @pl.when(sid == _SC_TILES - 1)
        def _():
            pltpu.sync_copy(acc.at[pl.ds(sid * stripe, last)],
                            out_hbm.at[cid, pl.ds(sid * stripe, last)])

    return k(ms, tei3, zeros_nd)


def _sc_pair_gather(h, row3d, col3d):
    n, d = h.shape
    nw, rows_per_tile, chunk = row3d.shape
    ep_pad = nw * rows_per_tile * chunk
    mesh = plsc.VectorSubcoreMesh(core_axis_name="c", subcore_axis_name="s")

    @functools.partial(
        pl.kernel,
        out_type=[jax.ShapeDtypeStruct((ep_pad, d), jnp.float32),
                  jax.ShapeDtypeStruct((ep_pad, d), jnp.float32)],
        mesh=mesh,
        scratch_types=[
            pltpu.VMEM((rows_per_tile, chunk), jnp.int32),
            pltpu.VMEM((rows_per_tile, chunk), jnp.int32),
            pltpu.VMEM((chunk, d), jnp.float32),
            pltpu.VMEM((chunk, d), jnp.float32),
            pltpu.SemaphoreType.DMA,
            pltpu.SemaphoreType.DMA,
        ],
    )
    def k(h_hbm, row_hbm, col_hbm, oa_hbm, ob_hbm,
          row_v, col_v, bufa, bufb, sema, semb):
        cid = lax.axis_index("c")
        sid = lax.axis_index("s")
        wid = cid * _SC_TILES + sid
        base = wid * rows_per_tile

        pltpu.sync_copy(row_hbm.at[wid], row_v)
        pltpu.sync_copy(col_hbm.at[wid], col_v)

        @pl.loop(0, rows_per_tile)
        def _(r):
            pltpu.async_copy(h_hbm.at[row_v.at[r]], bufa, sema)
            pltpu.async_copy(h_hbm.at[col_v.at[r]], bufb, semb)
            pltpu.make_async_copy(h_hbm.at[row_v.at[r]], bufa, sema).wait()
            pltpu.make_async_copy(h_hbm.at[col_v.at[r]], bufb, semb).wait()
            pltpu.sync_copy(bufa, oa_hbm.at[pl.ds((base + r) * chunk, chunk)])
            pltpu.sync_copy(bufb, ob_hbm.at[pl.ds((base + r) * chunk, chunk)])

    return k(h, row3d, col3d)


def kernel(x, edge_index, relations, concs, train_edge_index,
           W1, b1, W2, b2, Wl1, bl1, Wl2, bl2, Wr, br):
    n, d = x.shape
    e = train_edge_index.shape[1]
    tei3 = train_edge_index.reshape(2, e // _CHUNK, _CHUNK)
    zeros_n = jnp.zeros((n,), jnp.float32)
    zeros_nd = jnp.zeros((n, d), jnp.float32)

    cnt = _sc_degree(tei3, zeros_n, n)
    dinv2 = jax.lax.rsqrt(1.0 + cnt[:n] + cnt[n:])[:, None]

    m1s = _tc_linear_scale(x, W1, b1[None, :], dinv2)
    S1 = _sc_segsum(m1s, tei3, zeros_nd)
    h1, m2s = _tc_combine_next(S1, m1s, dinv2, W2, b2)
    S2 = _sc_segsum(m2s, tei3, zeros_nd)
    h2 = _tc_final_combine(S2, m2s, dinv2, h1)

    ep = edge_index.shape[0]
    gchunk = 64
    half = _NW * 25 * gchunk
    ep_pad = 2 * half
    pad = ep_pad - ep
    pad_idx = jnp.arange(pad, dtype=edge_index.dtype)
    rowp = jnp.concatenate([edge_index[:, 0], pad_idx])
    colp = jnp.concatenate([edge_index[:, 1], pad_idx])
    zlp = jnp.concatenate(
        [concs[:, 0] * concs[:, 1], jnp.zeros((pad,), jnp.float32)])[:, None]
    relp = jnp.concatenate(
        [relations.astype(jnp.float32), jnp.zeros((pad,), jnp.float32)])[:, None]
    outs = []
    for hix in range(2):
        lo, hi = hix * half, (hix + 1) * half
        row3d = rowp[lo:hi].reshape(_NW, 25, gchunk)
        col3d = colp[lo:hi].reshape(_NW, 25, gchunk)
        hr, hc = _sc_pair_gather(h2, row3d, col3d)
        outs.append(_tc_head(hr, hc, zlp[lo:hi], relp[lo:hi],
                             Wl1, bl1, Wl2, bl2, Wr, br))
    return jnp.concatenate(outs)[:ep, None]


# speedup vs baseline: 1.1881x; 1.1881x over previous
import functools
import jax
import jax.numpy as jnp
from jax import lax
from jax.experimental import pallas as pl
from jax.experimental.pallas import tpu as pltpu, tpu_sc as plsc

N_ROWS_BLK = 1000
E_BLK = 1024
_TC_PAR = pltpu.CompilerParams(dimension_semantics=("parallel",))


def _k1_body(x_ref, w_ref, b_ref, dinv_ref, o_ref):
    m = jnp.dot(x_ref[...], w_ref[...], preferred_element_type=jnp.float32)
    o_ref[...] = (m + b_ref[...]) * dinv_ref[...]


def _tc_linear_scale(x, W, b, dinv2):
    n, d = x.shape
    grid = (n // N_ROWS_BLK,)
    return pl.pallas_call(
        _k1_body,
        grid=grid,
        compiler_params=_TC_PAR,
        in_specs=[
            pl.BlockSpec((N_ROWS_BLK, d), lambda i: (i, 0)),
            pl.BlockSpec((d, d), lambda i: (0, 0)),
            pl.BlockSpec((1, d), lambda i: (0, 0)),
            pl.BlockSpec((N_ROWS_BLK, 1), lambda i: (i, 0)),
        ],
        out_specs=pl.BlockSpec((N_ROWS_BLK, d), lambda i: (i, 0)),
        out_shape=jax.ShapeDtypeStruct((n, d), jnp.float32),
    )(x, W, b, dinv2)


def _k2_body(s_ref, m_ref, dinv_ref, w_ref, b_ref, h_ref, o_ref):
    ssum = s_ref[0] + s_ref[1] + m_ref[...]
    h1 = jnp.maximum(ssum * dinv_ref[...], 0.0)
    h_ref[...] = h1
    m2 = jnp.dot(h1, w_ref[...], preferred_element_type=jnp.float32)
    o_ref[...] = (m2 + b_ref[...]) * dinv_ref[...]


def _tc_combine_next(S, m1s, dinv2, W2, b2):
    n, d = m1s.shape
    grid = (n // N_ROWS_BLK,)
    return pl.pallas_call(
        _k2_body,
        grid=grid,
        compiler_params=_TC_PAR,
        in_specs=[
            pl.BlockSpec((2, N_ROWS_BLK, d), lambda i: (0, i, 0)),
            pl.BlockSpec((N_ROWS_BLK, d), lambda i: (i, 0)),
            pl.BlockSpec((N_ROWS_BLK, 1), lambda i: (i, 0)),
            pl.BlockSpec((d, d), lambda i: (0, 0)),
            pl.BlockSpec((1, d), lambda i: (0, 0)),
        ],
        out_specs=[
            pl.BlockSpec((N_ROWS_BLK, d), lambda i: (i, 0)),
            pl.BlockSpec((N_ROWS_BLK, d), lambda i: (i, 0)),
        ],
        out_shape=[
            jax.ShapeDtypeStruct((n, d), jnp.float32),
            jax.ShapeDtypeStruct((n, d), jnp.float32),
        ],
    )(S, m1s, dinv2, W2, b2[None, :])


def _k3_body(s_ref, m_ref, dinv_ref, h1_ref, o_ref):
    ssum = s_ref[0] + s_ref[1] + m_ref[...]
    o_ref[...] = jnp.maximum(ssum * dinv_ref[...], 0.0) + h1_ref[...]


def _tc_final_combine(S, m2s, dinv2, h1):
    n, d = m2s.shape
    grid = (n // N_ROWS_BLK,)
    return pl.pallas_call(
        _k3_body,
        grid=grid,
        compiler_params=_TC_PAR,
        in_specs=[
            pl.BlockSpec((2, N_ROWS_BLK, d), lambda i: (0, i, 0)),
            pl.BlockSpec((N_ROWS_BLK, d), lambda i: (i, 0)),
            pl.BlockSpec((N_ROWS_BLK, 1), lambda i: (i, 0)),
            pl.BlockSpec((N_ROWS_BLK, d), lambda i: (i, 0)),
        ],
        out_specs=pl.BlockSpec((N_ROWS_BLK, d), lambda i: (i, 0)),
        out_shape=jax.ShapeDtypeStruct((n, d), jnp.float32),
    )(S, m2s, dinv2, h1)


def _head_body(hr_ref, hc_ref, zl_ref, rel_ref, w1_ref, w1l_ref, b1_ref,
               w2_ref, b2_ref, lanes_ref, wr2d_ref, brc_ref, ones_ref, o_ref):
    z = (hr_ref[...] * hc_ref[...]).astype(jnp.bfloat16)
    zl = zl_ref[...]
    a = jnp.dot(z, w1_ref[...], preferred_element_type=jnp.float32)
    a = a + zl * w1l_ref[...] + b1_ref[...]
    a = jnp.maximum(a, 0.0).astype(jnp.bfloat16)
    a = jnp.dot(a, w2_ref[...], preferred_element_type=jnp.float32) + b2_ref[...]
    a = jnp.maximum(a, 0.0)
    onehot = jnp.where(rel_ref[...] == lanes_ref[...], 1.0, 0.0)
    w = jnp.dot(onehot, wr2d_ref[...], preferred_element_type=jnp.float32)
    s1 = jnp.dot(a * w, ones_ref[...], preferred_element_type=jnp.float32)
    s2 = jnp.dot(onehot, brc_ref[...], preferred_element_type=jnp.float32)
    o_ref[...] = (s1 + s2)[:, 0]


def _tc_head(hr, hc, zlast2, rel2, Wl1, bl1, Wl2, bl2, Wr, br):
    e, d = hr.shape
    w1m = Wl1[:d].astype(jnp.bfloat16)
    w1l = Wl1[d:d + 1]
    lanes = jnp.arange(4, dtype=jnp.float32)[None, :]
    wr2d = Wr[:, :, 0]
    brc = br
    ones64 = jnp.ones((64, 1), jnp.float32)
    grid = (e // E_BLK,)
    return pl.pallas_call(
        _head_body,
        grid=grid,
        compiler_params=_TC_PAR,
        in_specs=[
            pl.BlockSpec((E_BLK, d), lambda i: (i, 0)),
            pl.BlockSpec((E_BLK, d), lambda i: (i, 0)),
            pl.BlockSpec((E_BLK, 1), lambda i: (i, 0)),
            pl.BlockSpec((E_BLK, 1), lambda i: (i, 0)),
            pl.BlockSpec((d, 256), lambda i: (0, 0)),
            pl.BlockSpec((1, 256), lambda i: (0, 0)),
            pl.BlockSpec((1, 256), lambda i: (0, 0)),
            pl.BlockSpec((256, 64), lambda i: (0, 0)),
            pl.BlockSpec((1, 64), lambda i: (0, 0)),
            pl.BlockSpec((1, 4), lambda i: (0, 0)),
            pl.BlockSpec((4, 64), lambda i: (0, 0)),
            pl.BlockSpec((4, 1), lambda i: (0, 0)),
            pl.BlockSpec((64, 1), lambda i: (0, 0)),
        ],
        out_specs=pl.BlockSpec((E_BLK,), lambda i: (i,)),
        out_shape=jax.ShapeDtypeStruct((e,), jnp.float32),
    )(hr, hc, zlast2, rel2, w1m, w1l, bl1[None, :], Wl2.astype(jnp.bfloat16),
      bl2[None, :], lanes, wr2d, brc, ones64)


_SC_CORES = 2
_SC_TILES = 16
_NW = _SC_CORES * _SC_TILES
_CHUNK = 125
_SEG = 16


def _sc_degree(tei3, zeros_n, n):
    rows_total, chunk = tei3.shape[1:]
    rows_per_tile = rows_total // _NW
    stripe = 624
    last = n - (_SC_TILES - 1) * stripe
    mesh = plsc.VectorSubcoreMesh(core_axis_name="c", subcore_axis_name="s")

    @functools.partial(
        pl.kernel,
        out_type=jax.ShapeDtypeStruct((_SC_CORES * n,), jnp.float32),
        mesh=mesh,
        scratch_types=[
            pltpu.VMEM((rows_per_tile, chunk), jnp.int32),
            pltpu.VMEM((128,), jnp.float32),
            pltpu.VMEM((640,), jnp.float32),
            pltpu.VMEM_SHARED((n,), jnp.float32),
        ],
    )
    def k(tei_hbm, z_hbm, out_hbm, dst_v, ones_v, zbuf, acc):
        del z_hbm
        cid = lax.axis_index("c")
        sid = lax.axis_index("s")
        wid = cid * _SC_TILES + sid
        dst_hbm = tei_hbm.at[1]

        @pl.loop(0, 128, step=16)
        def _(i):
            ones_v[pl.ds(i, 16)] = jnp.full((16,), 1.0, jnp.float32)

        @pl.loop(0, 640, step=16)
        def _(i):
            zbuf[pl.ds(i, 16)] = jnp.zeros((16,), jnp.float32)

        @pl.when(sid < _SC_TILES - 1)
        def _():
            pltpu.sync_copy(zbuf.at[pl.ds(0, stripe)],
                            acc.at[pl.ds(sid * stripe, stripe)])

        @pl.when(sid == _SC_TILES - 1)
        def _():
            pltpu.sync_copy(zbuf.at[pl.ds(0, last)],
                            acc.at[pl.ds(sid * stripe, last)])

        pltpu.sync_copy(dst_hbm.at[pl.ds(wid * rows_per_tile, rows_per_tile)],
                        dst_v)
        plsc.subcore_barrier()

        @pl.loop(0, rows_per_tile)
        def _(r):
            pltpu.sync_copy(ones_v.at[pl.ds(0, chunk)],
                            acc.at[dst_v.at[r]], add=True)

        plsc.subcore_barrier()

        @pl.when(sid < _SC_TILES - 1)
        def _():
            pltpu.sync_copy(acc.at[pl.ds(sid * stripe, stripe)],
                            zbuf.at[pl.ds(0, stripe)])
            pltpu.sync_copy(zbuf.at[pl.ds(0, stripe)],
                            out_hbm.at[pl.ds(cid * n + sid * stripe, stripe)])

        @pl.when(sid == _SC_TILES - 1)
        def _():
            pltpu.sync_copy(acc.at[pl.ds(sid * stripe, last)],
                            zbuf.at[pl.ds(0, last)])
            pltpu.sync_copy(zbuf.at[pl.ds(0, last)],
                            out_hbm.at[pl.ds(cid * n + sid * stripe, last)])

    return k(tei3, zeros_n)


def _sc_segsum(ms, tei3, zeros_nd):
    n, d = ms.shape
    rows_total, chunk = tei3.shape[1:]
    rows_per_tile = rows_total // _NW
    stripe = 624
    last = n - (_SC_TILES - 1) * stripe
    mesh = plsc.VectorSubcoreMesh(core_axis_name="c", subcore_axis_name="s")

    @functools.partial(
        pl.kernel,
        out_type=jax.ShapeDtypeStruct((_SC_CORES, n, d), jnp.float32),
        mesh=mesh,
        scratch_types=[
            pltpu.VMEM((_SEG, chunk), jnp.int32),
            pltpu.VMEM((_SEG, chunk), jnp.int32),
            pltpu.VMEM((chunk, d), jnp.float32),
            pltpu.VMEM((chunk, d), jnp.float32),
            pltpu.VMEM_SHARED((n, d), jnp.float32),
            pltpu.SemaphoreType.DMA,
            pltpu.SemaphoreType.DMA,
        ],
    )
    def k(ms_hbm, tei_hbm, z_hbm, out_hbm,
          src_v, dst_v, buf0, buf1, acc, sem0, sem1):
        cid = lax.axis_index("c")
        sid = lax.axis_index("s")
        wid = cid * _SC_TILES + sid
        src_hbm = tei_hbm.at[0]
        dst_hbm = tei_hbm.at[1]

        @pl.when(sid < _SC_TILES - 1)
        def _():
            pltpu.sync_copy(z_hbm.at[pl.ds(sid * stripe, stripe)],
                            acc.at[pl.ds(sid * stripe, stripe)])

        @pl.when(sid == _SC_TILES - 1)
        def _():
            pltpu.sync_copy(z_hbm.at[pl.ds(sid * stripe, last)],
                            acc.at[pl.ds(sid * stripe, last)])

        plsc.subcore_barrier()

        @pl.loop(0, rows_per_tile, step=_SEG)
        def _(seg):
            base = wid * rows_per_tile + seg
            pltpu.sync_copy(src_hbm.at[pl.ds(base, _SEG)], src_v)
            pltpu.sync_copy(dst_hbm.at[pl.ds(base, _SEG)], dst_v)
            pltpu.async_copy(ms_hbm.at[src_v.at[0]], buf0, sem0)

            @pl.loop(0, _SEG, step=2)
            def _(r):
                pltpu.async_copy(ms_hbm.at[src_v.at[r + 1]], buf1, sem1)
                pltpu.make_async_copy(ms_hbm.at[src_v.at[r]], buf0, sem0).wait()
                pltpu.sync_copy(buf0, acc.at[dst_v.at[r]], add=True)

                @pl.when(r + 2 < _SEG)
                def _():
                    pltpu.async_copy(ms_hbm.at[src_v.at[r + 2]], buf0, sem0)

                pltpu.make_async_copy(ms_hbm.at[src_v.at[r + 1]], buf1, sem1).wait()
                pltpu.sync_copy(buf1, acc.at[dst_v.at[r + 1]], add=True)

        plsc.subcore_barrier()

        @pl.when(sid < _SC_TILES - 1)
        def _():
            pltpu.sync_copy(acc.at[pl.ds(sid * stripe, stripe)],
                            out_hbm.at[cid, pl.ds(sid * stripe, stripe)])

        @pl.when(sid == _SC_TILES - 1)
        def _():
            pltpu.sync_copy(acc.at[pl.ds(sid * stripe, last)],
                            out_hbm.at[cid, pl.ds(sid * stripe, last)])

    return k(ms, tei3, zeros_nd)


def _sc_pair_gather(h, row3d, col3d):
    n, d = h.shape
    nw, rows_per_tile, chunk = row3d.shape
    ep_pad = nw * rows_per_tile * chunk
    mesh = plsc.VectorSubcoreMesh(core_axis_name="c", subcore_axis_name="s")

    @functools.partial(
        pl.kernel,
        out_type=[jax.ShapeDtypeStruct((ep_pad, d), jnp.float32),
                  jax.ShapeDtypeStruct((ep_pad, d), jnp.float32)],
        mesh=mesh,
        scratch_types=[
            pltpu.VMEM((rows_per_tile, chunk), jnp.int32),
            pltpu.VMEM((rows_per_tile, chunk), jnp.int32),
            pltpu.VMEM((chunk, d), jnp.float32),
            pltpu.VMEM((chunk, d), jnp.float32),
            pltpu.SemaphoreType.DMA,
            pltpu.SemaphoreType.DMA,
        ],
    )
    def k(h_hbm, row_hbm, col_hbm, oa_hbm, ob_hbm,
          row_v, col_v, bufa, bufb, sema, semb):
        cid = lax.axis_index("c")
        sid = lax.axis_index("s")
        wid = cid * _SC_TILES + sid
        base = wid * rows_per_tile

        pltpu.sync_copy(row_hbm.at[wid], row_v)
        pltpu.sync_copy(col_hbm.at[wid], col_v)

        @pl.loop(0, rows_per_tile)
        def _(r):
            pltpu.async_copy(h_hbm.at[row_v.at[r]], bufa, sema)
            pltpu.async_copy(h_hbm.at[col_v.at[r]], bufb, semb)
            pltpu.make_async_copy(h_hbm.at[row_v.at[r]], bufa, sema).wait()
            pltpu.make_async_copy(h_hbm.at[col_v.at[r]], bufb, semb).wait()
            pltpu.sync_copy(bufa, oa_hbm.at[pl.ds((base + r) * chunk, chunk)])
            pltpu.sync_copy(bufb, ob_hbm.at[pl.ds((base + r) * chunk, chunk)])

    return k(h, row3d, col3d)


def kernel(x, edge_index, relations, concs, train_edge_index,
           W1, b1, W2, b2, Wl1, bl1, Wl2, bl2, Wr, br):
    n, d = x.shape
    e = train_edge_index.shape[1]
    tei3 = train_edge_index.reshape(2, e // _CHUNK, _CHUNK)
    zeros_n = jnp.zeros((n,), jnp.float32)
    zeros_nd = jnp.zeros((n, d), jnp.float32)

    cnt = _sc_degree(tei3, zeros_n, n)
    dinv2 = jax.lax.rsqrt(1.0 + cnt[:n] + cnt[n:])[:, None]

    m1s = _tc_linear_scale(x, W1, b1[None, :], dinv2)
    S1 = _sc_segsum(m1s, tei3, zeros_nd)
    h1, m2s = _tc_combine_next(S1, m1s, dinv2, W2, b2)
    S2 = _sc_segsum(m2s, tei3, zeros_nd)
    h2 = _tc_final_combine(S2, m2s, dinv2, h1)

    ep = edge_index.shape[0]
    gchunk = 128
    rpt = -(-ep // (_NW * gchunk))
    ep_pad = _NW * rpt * gchunk
    pad = ep_pad - ep
    pad_idx = jnp.arange(pad, dtype=edge_index.dtype)
    row3d = jnp.concatenate(
        [edge_index[:, 0], pad_idx]).reshape(_NW, rpt, gchunk)
    col3d = jnp.concatenate(
        [edge_index[:, 1], pad_idx]).reshape(_NW, rpt, gchunk)
    hr, hc = _sc_pair_gather(h2, row3d, col3d)
    zlast2 = jnp.concatenate(
        [concs[:, 0] * concs[:, 1], jnp.zeros((pad,), jnp.float32)])[:, None]
    rel2 = jnp.concatenate(
        [relations.astype(jnp.float32), jnp.zeros((pad,), jnp.float32)])[:, None]
    out = _tc_head(hr, hc, zlast2, rel2, Wl1, bl1, Wl2, bl2, Wr, br)
    return out[:ep, None]


# speedup vs baseline: 1.2561x; 1.0573x over previous
import functools
import jax
import jax.numpy as jnp
from jax import lax
from jax.experimental import pallas as pl
from jax.experimental.pallas import tpu as pltpu, tpu_sc as plsc

N_ROWS_BLK = 1000
E_BLK = 2048
_TC_PAR = pltpu.CompilerParams(dimension_semantics=("parallel",))


def _k1_body(x_ref, w_ref, b_ref, dinv_ref, o_ref):
    m = jnp.dot(x_ref[...], w_ref[...], preferred_element_type=jnp.float32)
    o_ref[...] = (m + b_ref[...]) * dinv_ref[...]


def _tc_linear_scale(x, W, b, dinv2):
    n, d = x.shape
    grid = (n // N_ROWS_BLK,)
    return pl.pallas_call(
        _k1_body,
        grid=grid,
        compiler_params=_TC_PAR,
        in_specs=[
            pl.BlockSpec((N_ROWS_BLK, d), lambda i: (i, 0)),
            pl.BlockSpec((d, d), lambda i: (0, 0)),
            pl.BlockSpec((1, d), lambda i: (0, 0)),
            pl.BlockSpec((N_ROWS_BLK, 1), lambda i: (i, 0)),
        ],
        out_specs=pl.BlockSpec((N_ROWS_BLK, d), lambda i: (i, 0)),
        out_shape=jax.ShapeDtypeStruct((n, d), jnp.float32),
    )(x, W, b, dinv2)


def _k2_body(s_ref, m_ref, dinv_ref, w_ref, b_ref, h_ref, o_ref):
    ssum = s_ref[0] + s_ref[1] + m_ref[...]
    h1 = jnp.maximum(ssum * dinv_ref[...], 0.0)
    h_ref[...] = h1
    m2 = jnp.dot(h1, w_ref[...], preferred_element_type=jnp.float32)
    o_ref[...] = (m2 + b_ref[...]) * dinv_ref[...]


def _tc_combine_next(S, m1s, dinv2, W2, b2):
    n, d = m1s.shape
    grid = (n // N_ROWS_BLK,)
    return pl.pallas_call(
        _k2_body,
        grid=grid,
        compiler_params=_TC_PAR,
        in_specs=[
            pl.BlockSpec((2, N_ROWS_BLK, d), lambda i: (0, i, 0)),
            pl.BlockSpec((N_ROWS_BLK, d), lambda i: (i, 0)),
            pl.BlockSpec((N_ROWS_BLK, 1), lambda i: (i, 0)),
            pl.BlockSpec((d, d), lambda i: (0, 0)),
            pl.BlockSpec((1, d), lambda i: (0, 0)),
        ],
        out_specs=[
            pl.BlockSpec((N_ROWS_BLK, d), lambda i: (i, 0)),
            pl.BlockSpec((N_ROWS_BLK, d), lambda i: (i, 0)),
        ],
        out_shape=[
            jax.ShapeDtypeStruct((n, d), jnp.float32),
            jax.ShapeDtypeStruct((n, d), jnp.float32),
        ],
    )(S, m1s, dinv2, W2, b2[None, :])


def _k3_body(s_ref, m_ref, dinv_ref, h1_ref, o_ref):
    ssum = s_ref[0] + s_ref[1] + m_ref[...]
    o_ref[...] = jnp.maximum(ssum * dinv_ref[...], 0.0) + h1_ref[...]


def _tc_final_combine(S, m2s, dinv2, h1):
    n, d = m2s.shape
    grid = (n // N_ROWS_BLK,)
    return pl.pallas_call(
        _k3_body,
        grid=grid,
        compiler_params=_TC_PAR,
        in_specs=[
            pl.BlockSpec((2, N_ROWS_BLK, d), lambda i: (0, i, 0)),
            pl.BlockSpec((N_ROWS_BLK, d), lambda i: (i, 0)),
            pl.BlockSpec((N_ROWS_BLK, 1), lambda i: (i, 0)),
            pl.BlockSpec((N_ROWS_BLK, d), lambda i: (i, 0)),
        ],
        out_specs=pl.BlockSpec((N_ROWS_BLK, d), lambda i: (i, 0)),
        out_shape=jax.ShapeDtypeStruct((n, d), jnp.float32),
    )(S, m2s, dinv2, h1)


def _head_body(hr_ref, hc_ref, zl_ref, rel_ref, w1_ref, w1l_ref, b1_ref,
               w2_ref, b2_ref, lanes_ref, wr2d_ref, brc_ref, ones_ref, o_ref):
    z = (hr_ref[...] * hc_ref[...]).astype(jnp.bfloat16)
    zl = zl_ref[...]
    a = jnp.dot(z, w1_ref[...], preferred_element_type=jnp.float32)
    a = a + zl * w1l_ref[...] + b1_ref[...]
    a = jnp.maximum(a, 0.0).astype(jnp.bfloat16)
    a = jnp.dot(a, w2_ref[...], preferred_element_type=jnp.float32) + b2_ref[...]
    a = jnp.maximum(a, 0.0)
    onehot = jnp.where(rel_ref[...] == lanes_ref[...], 1.0, 0.0)
    w = jnp.dot(onehot, wr2d_ref[...], preferred_element_type=jnp.float32)
    s1 = jnp.dot(a * w, ones_ref[...], preferred_element_type=jnp.float32)
    s2 = jnp.dot(onehot, brc_ref[...], preferred_element_type=jnp.float32)
    o_ref[...] = (s1 + s2)[:, 0]


def _tc_head(hr, hc, zlast2, rel2, Wl1, bl1, Wl2, bl2, Wr, br):
    e, d = hr.shape
    w1m = Wl1[:d].astype(jnp.bfloat16)
    w1l = Wl1[d:d + 1]
    lanes = jnp.arange(4, dtype=jnp.float32)[None, :]
    wr2d = Wr[:, :, 0]
    brc = br
    ones64 = jnp.ones((64, 1), jnp.float32)
    grid = (e // E_BLK,)
    return pl.pallas_call(
        _head_body,
        grid=grid,
        compiler_params=_TC_PAR,
        in_specs=[
            pl.BlockSpec((E_BLK, d), lambda i: (i, 0)),
            pl.BlockSpec((E_BLK, d), lambda i: (i, 0)),
            pl.BlockSpec((E_BLK, 1), lambda i: (i, 0)),
            pl.BlockSpec((E_BLK, 1), lambda i: (i, 0)),
            pl.BlockSpec((d, 256), lambda i: (0, 0)),
            pl.BlockSpec((1, 256), lambda i: (0, 0)),
            pl.BlockSpec((1, 256), lambda i: (0, 0)),
            pl.BlockSpec((256, 64), lambda i: (0, 0)),
            pl.BlockSpec((1, 64), lambda i: (0, 0)),
            pl.BlockSpec((1, 4), lambda i: (0, 0)),
            pl.BlockSpec((4, 64), lambda i: (0, 0)),
            pl.BlockSpec((4, 1), lambda i: (0, 0)),
            pl.BlockSpec((64, 1), lambda i: (0, 0)),
        ],
        out_specs=pl.BlockSpec((E_BLK,), lambda i: (i,)),
        out_shape=jax.ShapeDtypeStruct((e,), jnp.float32),
    )(hr, hc, zlast2, rel2, w1m, w1l, bl1[None, :], Wl2.astype(jnp.bfloat16),
      bl2[None, :], lanes, wr2d, brc, ones64)


_SC_CORES = 2
_SC_TILES = 16
_NW = _SC_CORES * _SC_TILES
_CHUNK = 125
_SEG = 16


def _sc_degree(tei3, zeros_n, n):
    rows_total, chunk = tei3.shape[1:]
    rows_per_tile = rows_total // _NW
    stripe = 624
    last = n - (_SC_TILES - 1) * stripe
    mesh = plsc.VectorSubcoreMesh(core_axis_name="c", subcore_axis_name="s")

    @functools.partial(
        pl.kernel,
        out_type=jax.ShapeDtypeStruct((_SC_CORES * n,), jnp.float32),
        mesh=mesh,
        scratch_types=[
            pltpu.VMEM((rows_per_tile, chunk), jnp.int32),
            pltpu.VMEM((128,), jnp.float32),
            pltpu.VMEM((640,), jnp.float32),
            pltpu.VMEM_SHARED((n,), jnp.float32),
        ],
    )
    def k(tei_hbm, z_hbm, out_hbm, dst_v, ones_v, zbuf, acc):
        del z_hbm
        cid = lax.axis_index("c")
        sid = lax.axis_index("s")
        wid = cid * _SC_TILES + sid
        dst_hbm = tei_hbm.at[1]

        @pl.loop(0, 128, step=16)
        def _(i):
            ones_v[pl.ds(i, 16)] = jnp.full((16,), 1.0, jnp.float32)

        @pl.loop(0, 640, step=16)
        def _(i):
            zbuf[pl.ds(i, 16)] = jnp.zeros((16,), jnp.float32)

        @pl.when(sid < _SC_TILES - 1)
        def _():
            pltpu.sync_copy(zbuf.at[pl.ds(0, stripe)],
                            acc.at[pl.ds(sid * stripe, stripe)])

        @pl.when(sid == _SC_TILES - 1)
        def _():
            pltpu.sync_copy(zbuf.at[pl.ds(0, last)],
                            acc.at[pl.ds(sid * stripe, last)])

        pltpu.sync_copy(dst_hbm.at[pl.ds(wid * rows_per_tile, rows_per_tile)],
                        dst_v)
        plsc.subcore_barrier()

        @pl.loop(0, rows_per_tile)
        def _(r):
            pltpu.sync_copy(ones_v.at[pl.ds(0, chunk)],
                            acc.at[dst_v.at[r]], add=True)

        plsc.subcore_barrier()

        @pl.when(sid < _SC_TILES - 1)
        def _():
            pltpu.sync_copy(acc.at[pl.ds(sid * stripe, stripe)],
                            zbuf.at[pl.ds(0, stripe)])
            pltpu.sync_copy(zbuf.at[pl.ds(0, stripe)],
                            out_hbm.at[pl.ds(cid * n + sid * stripe, stripe)])

        @pl.when(sid == _SC_TILES - 1)
        def _():
            pltpu.sync_copy(acc.at[pl.ds(sid * stripe, last)],
                            zbuf.at[pl.ds(0, last)])
            pltpu.sync_copy(zbuf.at[pl.ds(0, last)],
                            out_hbm.at[pl.ds(cid * n + sid * stripe, last)])

    return k(tei3, zeros_n)


def _sc_segsum(ms, tei3, zeros_nd):
    n, d = ms.shape
    rows_total, chunk = tei3.shape[1:]
    rows_per_tile = rows_total // _NW
    stripe = 624
    last = n - (_SC_TILES - 1) * stripe
    mesh = plsc.VectorSubcoreMesh(core_axis_name="c", subcore_axis_name="s")

    @functools.partial(
        pl.kernel,
        out_type=jax.ShapeDtypeStruct((_SC_CORES, n, d), jnp.float32),
        mesh=mesh,
        scratch_types=[
            pltpu.VMEM((_SEG, chunk), jnp.int32),
            pltpu.VMEM((_SEG, chunk), jnp.int32),
            pltpu.VMEM((chunk, d), jnp.float32),
            pltpu.VMEM((chunk, d), jnp.float32),
            pltpu.VMEM_SHARED((n, d), jnp.float32),
            pltpu.SemaphoreType.DMA,
            pltpu.SemaphoreType.DMA,
        ],
    )
    def k(ms_hbm, tei_hbm, z_hbm, out_hbm,
          src_v, dst_v, buf0, buf1, acc, sem0, sem1):
        cid = lax.axis_index("c")
        sid = lax.axis_index("s")
        wid = cid * _SC_TILES + sid
        src_hbm = tei_hbm.at[0]
        dst_hbm = tei_hbm.at[1]

        @pl.when(sid < _SC_TILES - 1)
        def _():
            pltpu.sync_copy(z_hbm.at[pl.ds(sid * stripe, stripe)],
                            acc.at[pl.ds(sid * stripe, stripe)])

        @pl.when(sid == _SC_TILES - 1)
        def _():
            pltpu.sync_copy(z_hbm.at[pl.ds(sid * stripe, last)],
                            acc.at[pl.ds(sid * stripe, last)])

        plsc.subcore_barrier()

        @pl.loop(0, rows_per_tile, step=_SEG)
        def _(seg):
            base = wid * rows_per_tile + seg
            pltpu.sync_copy(src_hbm.at[pl.ds(base, _SEG)], src_v)
            pltpu.sync_copy(dst_hbm.at[pl.ds(base, _SEG)], dst_v)
            pltpu.async_copy(ms_hbm.at[src_v.at[0]], buf0, sem0)

            @pl.loop(0, _SEG, step=2)
            def _(r):
                pltpu.async_copy(ms_hbm.at[src_v.at[r + 1]], buf1, sem1)
                pltpu.make_async_copy(ms_hbm.at[src_v.at[r]], buf0, sem0).wait()
                pltpu.sync_copy(buf0, acc.at[dst_v.at[r]], add=True)

                @pl.when(r + 2 < _SEG)
                def _():
                    pltpu.async_copy(ms_hbm.at[src_v.at[r + 2]], buf0, sem0)

                pltpu.make_async_copy(ms_hbm.at[src_v.at[r + 1]], buf1, sem1).wait()
                pltpu.sync_copy(buf1, acc.at[dst_v.at[r + 1]], add=True)

        plsc.subcore_barrier()

        @pl.when(sid < _SC_TILES - 1)
        def _():
            pltpu.sync_copy(acc.at[pl.ds(sid * stripe, stripe)],
                            out_hbm.at[cid, pl.ds(sid * stripe, stripe)])

        @pl.when(sid == _SC_TILES - 1)
        def _():
            pltpu.sync_copy(acc.at[pl.ds(sid * stripe, last)],
                            out_hbm.at[cid, pl.ds(sid * stripe, last)])

    return k(ms, tei3, zeros_nd)


def _sc_pair_gather(h, row3d, col3d):
    n, d = h.shape
    nw, rows_per_tile, chunk = row3d.shape
    ep_pad = nw * rows_per_tile * chunk
    mesh = plsc.VectorSubcoreMesh(core_axis_name="c", subcore_axis_name="s")

    @functools.partial(
        pl.kernel,
        out_type=[jax.ShapeDtypeStruct((ep_pad, d), jnp.float32),
                  jax.ShapeDtypeStruct((ep_pad, d), jnp.float32)],
        mesh=mesh,
        scratch_types=[
            pltpu.VMEM((rows_per_tile, chunk), jnp.int32),
            pltpu.VMEM((rows_per_tile, chunk), jnp.int32),
            pltpu.VMEM((chunk, d), jnp.float32),
            pltpu.VMEM((chunk, d), jnp.float32),
            pltpu.SemaphoreType.DMA,
            pltpu.SemaphoreType.DMA,
        ],
    )
    def k(h_hbm, row_hbm, col_hbm, oa_hbm, ob_hbm,
          row_v, col_v, bufa, bufb, sema, semb):
        cid = lax.axis_index("c")
        sid = lax.axis_index("s")
        wid = cid * _SC_TILES + sid
        base = wid * rows_per_tile

        pltpu.sync_copy(row_hbm.at[wid], row_v)
        pltpu.sync_copy(col_hbm.at[wid], col_v)

        @pl.loop(0, rows_per_tile)
        def _(r):
            pltpu.async_copy(h_hbm.at[row_v.at[r]], bufa, sema)
            pltpu.async_copy(h_hbm.at[col_v.at[r]], bufb, semb)
            pltpu.make_async_copy(h_hbm.at[row_v.at[r]], bufa, sema).wait()
            pltpu.make_async_copy(h_hbm.at[col_v.at[r]], bufb, semb).wait()
            pltpu.sync_copy(bufa, oa_hbm.at[pl.ds((base + r) * chunk, chunk)])
            pltpu.sync_copy(bufb, ob_hbm.at[pl.ds((base + r) * chunk, chunk)])

    return k(h, row3d, col3d)


def kernel(x, edge_index, relations, concs, train_edge_index,
           W1, b1, W2, b2, Wl1, bl1, Wl2, bl2, Wr, br):
    n, d = x.shape
    e = train_edge_index.shape[1]
    tei3 = train_edge_index.reshape(2, e // _CHUNK, _CHUNK)
    zeros_n = jnp.zeros((n,), jnp.float32)
    zeros_nd = jnp.zeros((n, d), jnp.float32)

    cnt = _sc_degree(tei3, zeros_n, n)
    dinv2 = jax.lax.rsqrt(1.0 + cnt[:n] + cnt[n:])[:, None]

    m1s = _tc_linear_scale(x, W1, b1[None, :], dinv2)
    S1 = _sc_segsum(m1s, tei3, zeros_nd)
    h1, m2s = _tc_combine_next(S1, m1s, dinv2, W2, b2)
    S2 = _sc_segsum(m2s, tei3, zeros_nd)
    h2 = _tc_final_combine(S2, m2s, dinv2, h1)

    ep = edge_index.shape[0]
    gchunk = 128
    rpt = -(-ep // (_NW * gchunk))
    ep_pad = _NW * rpt * gchunk
    pad = ep_pad - ep
    pad_idx = jnp.arange(pad, dtype=edge_index.dtype)
    row3d = jnp.concatenate(
        [edge_index[:, 0], pad_idx]).reshape(_NW, rpt, gchunk)
    col3d = jnp.concatenate(
        [edge_index[:, 1], pad_idx]).reshape(_NW, rpt, gchunk)
    hr, hc = _sc_pair_gather(h2, row3d, col3d)
    zlast2 = jnp.concatenate(
        [concs[:, 0] * concs[:, 1], jnp.zeros((pad,), jnp.float32)])[:, None]
    rel2 = jnp.concatenate(
        [relations.astype(jnp.float32), jnp.zeros((pad,), jnp.float32)])[:, None]
    out = _tc_head(hr, hc, zlast2, rel2, Wl1, bl1, Wl2, bl2, Wr, br)
    return out[:ep, None]


# speedup vs baseline: 1.2816x; 1.0203x over previous
import functools
import jax
import jax.numpy as jnp
from jax import lax
from jax.experimental import pallas as pl
from jax.experimental.pallas import tpu as pltpu, tpu_sc as plsc

N_ROWS_BLK = 1000
E_BLK = 4096
_TC_PAR = pltpu.CompilerParams(dimension_semantics=("parallel",))


def _k1_body(x_ref, w_ref, b_ref, dinv_ref, o_ref):
    m = jnp.dot(x_ref[...], w_ref[...], preferred_element_type=jnp.float32)
    o_ref[...] = (m + b_ref[...]) * dinv_ref[...]


def _tc_linear_scale(x, W, b, dinv2):
    n, d = x.shape
    grid = (n // N_ROWS_BLK,)
    return pl.pallas_call(
        _k1_body,
        grid=grid,
        compiler_params=_TC_PAR,
        in_specs=[
            pl.BlockSpec((N_ROWS_BLK, d), lambda i: (i, 0)),
            pl.BlockSpec((d, d), lambda i: (0, 0)),
            pl.BlockSpec((1, d), lambda i: (0, 0)),
            pl.BlockSpec((N_ROWS_BLK, 1), lambda i: (i, 0)),
        ],
        out_specs=pl.BlockSpec((N_ROWS_BLK, d), lambda i: (i, 0)),
        out_shape=jax.ShapeDtypeStruct((n, d), jnp.float32),
    )(x, W, b, dinv2)


def _k2_body(s_ref, m_ref, dinv_ref, w_ref, b_ref, h_ref, o_ref):
    ssum = s_ref[0] + s_ref[1] + m_ref[...]
    h1 = jnp.maximum(ssum * dinv_ref[...], 0.0)
    h_ref[...] = h1
    m2 = jnp.dot(h1, w_ref[...], preferred_element_type=jnp.float32)
    o_ref[...] = (m2 + b_ref[...]) * dinv_ref[...]


def _tc_combine_next(S, m1s, dinv2, W2, b2):
    n, d = m1s.shape
    grid = (n // N_ROWS_BLK,)
    return pl.pallas_call(
        _k2_body,
        grid=grid,
        compiler_params=_TC_PAR,
        in_specs=[
            pl.BlockSpec((2, N_ROWS_BLK, d), lambda i: (0, i, 0)),
            pl.BlockSpec((N_ROWS_BLK, d), lambda i: (i, 0)),
            pl.BlockSpec((N_ROWS_BLK, 1), lambda i: (i, 0)),
            pl.BlockSpec((d, d), lambda i: (0, 0)),
            pl.BlockSpec((1, d), lambda i: (0, 0)),
        ],
        out_specs=[
            pl.BlockSpec((N_ROWS_BLK, d), lambda i: (i, 0)),
            pl.BlockSpec((N_ROWS_BLK, d), lambda i: (i, 0)),
        ],
        out_shape=[
            jax.ShapeDtypeStruct((n, d), jnp.float32),
            jax.ShapeDtypeStruct((n, d), jnp.float32),
        ],
    )(S, m1s, dinv2, W2, b2[None, :])


def _k3_body(s_ref, m_ref, dinv_ref, h1_ref, o_ref):
    ssum = s_ref[0] + s_ref[1] + m_ref[...]
    o_ref[...] = jnp.maximum(ssum * dinv_ref[...], 0.0) + h1_ref[...]


def _tc_final_combine(S, m2s, dinv2, h1):
    n, d = m2s.shape
    grid = (n // N_ROWS_BLK,)
    return pl.pallas_call(
        _k3_body,
        grid=grid,
        compiler_params=_TC_PAR,
        in_specs=[
            pl.BlockSpec((2, N_ROWS_BLK, d), lambda i: (0, i, 0)),
            pl.BlockSpec((N_ROWS_BLK, d), lambda i: (i, 0)),
            pl.BlockSpec((N_ROWS_BLK, 1), lambda i: (i, 0)),
            pl.BlockSpec((N_ROWS_BLK, d), lambda i: (i, 0)),
        ],
        out_specs=pl.BlockSpec((N_ROWS_BLK, d), lambda i: (i, 0)),
        out_shape=jax.ShapeDtypeStruct((n, d), jnp.float32),
    )(S, m2s, dinv2, h1)


def _head_body(hr_ref, hc_ref, zl_ref, rel_ref, w1_ref, w1l_ref, b1_ref,
               w2_ref, b2_ref, lanes_ref, wr2d_ref, brc_ref, ones_ref, o_ref):
    z = (hr_ref[...] * hc_ref[...]).astype(jnp.bfloat16)
    zl = zl_ref[...]
    a = jnp.dot(z, w1_ref[...], preferred_element_type=jnp.float32)
    a = a + zl * w1l_ref[...] + b1_ref[...]
    a = jnp.maximum(a, 0.0).astype(jnp.bfloat16)
    a = jnp.dot(a, w2_ref[...], preferred_element_type=jnp.float32) + b2_ref[...]
    a = jnp.maximum(a, 0.0)
    onehot = jnp.where(rel_ref[...] == lanes_ref[...], 1.0, 0.0)
    w = jnp.dot(onehot, wr2d_ref[...], preferred_element_type=jnp.float32)
    s1 = jnp.dot(a * w, ones_ref[...], preferred_element_type=jnp.float32)
    s2 = jnp.dot(onehot, brc_ref[...], preferred_element_type=jnp.float32)
    o_ref[...] = (s1 + s2)[:, 0]


def _tc_head(hr, hc, zlast2, rel2, Wl1, bl1, Wl2, bl2, Wr, br):
    e, d = hr.shape
    w1m = Wl1[:d].astype(jnp.bfloat16)
    w1l = Wl1[d:d + 1]
    lanes = jnp.arange(4, dtype=jnp.float32)[None, :]
    wr2d = Wr[:, :, 0]
    brc = br
    ones64 = jnp.ones((64, 1), jnp.float32)
    grid = (e // E_BLK,)
    return pl.pallas_call(
        _head_body,
        grid=grid,
        compiler_params=_TC_PAR,
        in_specs=[
            pl.BlockSpec((E_BLK, d), lambda i: (i, 0)),
            pl.BlockSpec((E_BLK, d), lambda i: (i, 0)),
            pl.BlockSpec((E_BLK, 1), lambda i: (i, 0)),
            pl.BlockSpec((E_BLK, 1), lambda i: (i, 0)),
            pl.BlockSpec((d, 256), lambda i: (0, 0)),
            pl.BlockSpec((1, 256), lambda i: (0, 0)),
            pl.BlockSpec((1, 256), lambda i: (0, 0)),
            pl.BlockSpec((256, 64), lambda i: (0, 0)),
            pl.BlockSpec((1, 64), lambda i: (0, 0)),
            pl.BlockSpec((1, 4), lambda i: (0, 0)),
            pl.BlockSpec((4, 64), lambda i: (0, 0)),
            pl.BlockSpec((4, 1), lambda i: (0, 0)),
            pl.BlockSpec((64, 1), lambda i: (0, 0)),
        ],
        out_specs=pl.BlockSpec((E_BLK,), lambda i: (i,)),
        out_shape=jax.ShapeDtypeStruct((e,), jnp.float32),
    )(hr, hc, zlast2, rel2, w1m, w1l, bl1[None, :], Wl2.astype(jnp.bfloat16),
      bl2[None, :], lanes, wr2d, brc, ones64)


_SC_CORES = 2
_SC_TILES = 16
_NW = _SC_CORES * _SC_TILES
_CHUNK = 125
_SEG = 16


def _sc_degree(tei3, zeros_n, n):
    rows_total, chunk = tei3.shape[1:]
    rows_per_tile = rows_total // _NW
    stripe = 624
    last = n - (_SC_TILES - 1) * stripe
    mesh = plsc.VectorSubcoreMesh(core_axis_name="c", subcore_axis_name="s")

    @functools.partial(
        pl.kernel,
        out_type=jax.ShapeDtypeStruct((_SC_CORES * n,), jnp.float32),
        mesh=mesh,
        scratch_types=[
            pltpu.VMEM((rows_per_tile, chunk), jnp.int32),
            pltpu.VMEM((128,), jnp.float32),
            pltpu.VMEM((640,), jnp.float32),
            pltpu.VMEM_SHARED((n,), jnp.float32),
        ],
    )
    def k(tei_hbm, z_hbm, out_hbm, dst_v, ones_v, zbuf, acc):
        del z_hbm
        cid = lax.axis_index("c")
        sid = lax.axis_index("s")
        wid = cid * _SC_TILES + sid
        dst_hbm = tei_hbm.at[1]

        @pl.loop(0, 128, step=16)
        def _(i):
            ones_v[pl.ds(i, 16)] = jnp.full((16,), 1.0, jnp.float32)

        @pl.loop(0, 640, step=16)
        def _(i):
            zbuf[pl.ds(i, 16)] = jnp.zeros((16,), jnp.float32)

        @pl.when(sid < _SC_TILES - 1)
        def _():
            pltpu.sync_copy(zbuf.at[pl.ds(0, stripe)],
                            acc.at[pl.ds(sid * stripe, stripe)])

        @pl.when(sid == _SC_TILES - 1)
        def _():
            pltpu.sync_copy(zbuf.at[pl.ds(0, last)],
                            acc.at[pl.ds(sid * stripe, last)])

        pltpu.sync_copy(dst_hbm.at[pl.ds(wid * rows_per_tile, rows_per_tile)],
                        dst_v)
        plsc.subcore_barrier()

        @pl.loop(0, rows_per_tile)
        def _(r):
            pltpu.sync_copy(ones_v.at[pl.ds(0, chunk)],
                            acc.at[dst_v.at[r]], add=True)

        plsc.subcore_barrier()

        @pl.when(sid < _SC_TILES - 1)
        def _():
            pltpu.sync_copy(acc.at[pl.ds(sid * stripe, stripe)],
                            zbuf.at[pl.ds(0, stripe)])
            pltpu.sync_copy(zbuf.at[pl.ds(0, stripe)],
                            out_hbm.at[pl.ds(cid * n + sid * stripe, stripe)])

        @pl.when(sid == _SC_TILES - 1)
        def _():
            pltpu.sync_copy(acc.at[pl.ds(sid * stripe, last)],
                            zbuf.at[pl.ds(0, last)])
            pltpu.sync_copy(zbuf.at[pl.ds(0, last)],
                            out_hbm.at[pl.ds(cid * n + sid * stripe, last)])

    return k(tei3, zeros_n)


def _sc_segsum(ms, tei3, zeros_nd):
    n, d = ms.shape
    rows_total, chunk = tei3.shape[1:]
    rows_per_tile = rows_total // _NW
    stripe = 624
    last = n - (_SC_TILES - 1) * stripe
    mesh = plsc.VectorSubcoreMesh(core_axis_name="c", subcore_axis_name="s")

    @functools.partial(
        pl.kernel,
        out_type=jax.ShapeDtypeStruct((_SC_CORES, n, d), jnp.float32),
        mesh=mesh,
        scratch_types=[
            pltpu.VMEM((_SEG, chunk), jnp.int32),
            pltpu.VMEM((_SEG, chunk), jnp.int32),
            pltpu.VMEM((chunk, d), jnp.float32),
            pltpu.VMEM((chunk, d), jnp.float32),
            pltpu.VMEM_SHARED((n, d), jnp.float32),
            pltpu.SemaphoreType.DMA,
            pltpu.SemaphoreType.DMA,
        ],
    )
    def k(ms_hbm, tei_hbm, z_hbm, out_hbm,
          src_v, dst_v, buf0, buf1, acc, sem0, sem1):
        cid = lax.axis_index("c")
        sid = lax.axis_index("s")
        wid = cid * _SC_TILES + sid
        src_hbm = tei_hbm.at[0]
        dst_hbm = tei_hbm.at[1]

        @pl.when(sid < _SC_TILES - 1)
        def _():
            pltpu.sync_copy(z_hbm.at[pl.ds(sid * stripe, stripe)],
                            acc.at[pl.ds(sid * stripe, stripe)])

        @pl.when(sid == _SC_TILES - 1)
        def _():
            pltpu.sync_copy(z_hbm.at[pl.ds(sid * stripe, last)],
                            acc.at[pl.ds(sid * stripe, last)])

        plsc.subcore_barrier()

        @pl.loop(0, rows_per_tile, step=_SEG)
        def _(seg):
            base = wid * rows_per_tile + seg
            pltpu.sync_copy(src_hbm.at[pl.ds(base, _SEG)], src_v)
            pltpu.sync_copy(dst_hbm.at[pl.ds(base, _SEG)], dst_v)
            pltpu.async_copy(ms_hbm.at[src_v.at[0]], buf0, sem0)

            @pl.loop(0, _SEG, step=2)
            def _(r):
                pltpu.async_copy(ms_hbm.at[src_v.at[r + 1]], buf1, sem1)
                pltpu.make_async_copy(ms_hbm.at[src_v.at[r]], buf0, sem0).wait()
                pltpu.sync_copy(buf0, acc.at[dst_v.at[r]], add=True)

                @pl.when(r + 2 < _SEG)
                def _():
                    pltpu.async_copy(ms_hbm.at[src_v.at[r + 2]], buf0, sem0)

                pltpu.make_async_copy(ms_hbm.at[src_v.at[r + 1]], buf1, sem1).wait()
                pltpu.sync_copy(buf1, acc.at[dst_v.at[r + 1]], add=True)

        plsc.subcore_barrier()

        @pl.when(sid < _SC_TILES - 1)
        def _():
            pltpu.sync_copy(acc.at[pl.ds(sid * stripe, stripe)],
                            out_hbm.at[cid, pl.ds(sid * stripe, stripe)])

        @pl.when(sid == _SC_TILES - 1)
        def _():
            pltpu.sync_copy(acc.at[pl.ds(sid * stripe, last)],
                            out_hbm.at[cid, pl.ds(sid * stripe, last)])

    return k(ms, tei3, zeros_nd)


def _sc_pair_gather(h, row3d, col3d):
    n, d = h.shape
    nw, rows_per_tile, chunk = row3d.shape
    ep_pad = nw * rows_per_tile * chunk
    mesh = plsc.VectorSubcoreMesh(core_axis_name="c", subcore_axis_name="s")

    @functools.partial(
        pl.kernel,
        out_type=[jax.ShapeDtypeStruct((ep_pad, d), jnp.float32),
                  jax.ShapeDtypeStruct((ep_pad, d), jnp.float32)],
        mesh=mesh,
        scratch_types=[
            pltpu.VMEM((rows_per_tile, chunk), jnp.int32),
            pltpu.VMEM((rows_per_tile, chunk), jnp.int32),
            pltpu.VMEM((chunk, d), jnp.float32),
            pltpu.VMEM((chunk, d), jnp.float32),
            pltpu.SemaphoreType.DMA,
            pltpu.SemaphoreType.DMA,
        ],
    )
    def k(h_hbm, row_hbm, col_hbm, oa_hbm, ob_hbm,
          row_v, col_v, bufa, bufb, sema, semb):
        cid = lax.axis_index("c")
        sid = lax.axis_index("s")
        wid = cid * _SC_TILES + sid
        base = wid * rows_per_tile

        pltpu.sync_copy(row_hbm.at[wid], row_v)
        pltpu.sync_copy(col_hbm.at[wid], col_v)

        @pl.loop(0, rows_per_tile)
        def _(r):
            pltpu.async_copy(h_hbm.at[row_v.at[r]], bufa, sema)
            pltpu.async_copy(h_hbm.at[col_v.at[r]], bufb, semb)
            pltpu.make_async_copy(h_hbm.at[row_v.at[r]], bufa, sema).wait()
            pltpu.make_async_copy(h_hbm.at[col_v.at[r]], bufb, semb).wait()
            pltpu.sync_copy(bufa, oa_hbm.at[pl.ds((base + r) * chunk, chunk)])
            pltpu.sync_copy(bufb, ob_hbm.at[pl.ds((base + r) * chunk, chunk)])

    return k(h, row3d, col3d)


def kernel(x, edge_index, relations, concs, train_edge_index,
           W1, b1, W2, b2, Wl1, bl1, Wl2, bl2, Wr, br):
    n, d = x.shape
    e = train_edge_index.shape[1]
    tei3 = train_edge_index.reshape(2, e // _CHUNK, _CHUNK)
    zeros_n = jnp.zeros((n,), jnp.float32)
    zeros_nd = jnp.zeros((n, d), jnp.float32)

    cnt = _sc_degree(tei3, zeros_n, n)
    dinv2 = jax.lax.rsqrt(1.0 + cnt[:n] + cnt[n:])[:, None]

    m1s = _tc_linear_scale(x, W1, b1[None, :], dinv2)
    S1 = _sc_segsum(m1s, tei3, zeros_nd)
    h1, m2s = _tc_combine_next(S1, m1s, dinv2, W2, b2)
    S2 = _sc_segsum(m2s, tei3, zeros_nd)
    h2 = _tc_final_combine(S2, m2s, dinv2, h1)

    ep = edge_index.shape[0]
    gchunk = 128
    rpt = -(-ep // (_NW * gchunk))
    ep_pad = _NW * rpt * gchunk
    pad = ep_pad - ep
    pad_idx = jnp.arange(pad, dtype=edge_index.dtype)
    row3d = jnp.concatenate(
        [edge_index[:, 0], pad_idx]).reshape(_NW, rpt, gchunk)
    col3d = jnp.concatenate(
        [edge_index[:, 1], pad_idx]).reshape(_NW, rpt, gchunk)
    hr, hc = _sc_pair_gather(h2, row3d, col3d)
    zlast2 = jnp.concatenate(
        [concs[:, 0] * concs[:, 1], jnp.zeros((pad,), jnp.float32)])[:, None]
    rel2 = jnp.concatenate(
        [relations.astype(jnp.float32), jnp.zeros((pad,), jnp.float32)])[:, None]
    out = _tc_head(hr, hc, zlast2, rel2, Wl1, bl1, Wl2, bl2, Wr, br)
    return out[:ep, None]


# speedup vs baseline: 1.2961x; 1.0113x over previous
import functools
import jax
import jax.numpy as jnp
from jax import lax
from jax.experimental import pallas as pl
from jax.experimental.pallas import tpu as pltpu, tpu_sc as plsc

N_ROWS_BLK = 2000
E_BLK = 5120
_TC_PAR = pltpu.CompilerParams(dimension_semantics=("parallel",))


def _k1_body(x_ref, w_ref, b_ref, dinv_ref, o_ref):
    m = jnp.dot(x_ref[...], w_ref[...], preferred_element_type=jnp.float32)
    o_ref[...] = (m + b_ref[...]) * dinv_ref[...]


def _tc_linear_scale(x, W, b, dinv2):
    n, d = x.shape
    grid = (n // N_ROWS_BLK,)
    return pl.pallas_call(
        _k1_body,
        grid=grid,
        compiler_params=_TC_PAR,
        in_specs=[
            pl.BlockSpec((N_ROWS_BLK, d), lambda i: (i, 0)),
            pl.BlockSpec((d, d), lambda i: (0, 0)),
            pl.BlockSpec((1, d), lambda i: (0, 0)),
            pl.BlockSpec((N_ROWS_BLK, 1), lambda i: (i, 0)),
        ],
        out_specs=pl.BlockSpec((N_ROWS_BLK, d), lambda i: (i, 0)),
        out_shape=jax.ShapeDtypeStruct((n, d), jnp.float32),
    )(x, W, b, dinv2)


def _k2_body(s_ref, m_ref, dinv_ref, w_ref, b_ref, h_ref, o_ref):
    ssum = s_ref[0] + s_ref[1] + m_ref[...]
    h1 = jnp.maximum(ssum * dinv_ref[...], 0.0)
    h_ref[...] = h1
    m2 = jnp.dot(h1, w_ref[...], preferred_element_type=jnp.float32)
    o_ref[...] = (m2 + b_ref[...]) * dinv_ref[...]


def _tc_combine_next(S, m1s, dinv2, W2, b2):
    n, d = m1s.shape
    grid = (n // N_ROWS_BLK,)
    return pl.pallas_call(
        _k2_body,
        grid=grid,
        compiler_params=_TC_PAR,
        in_specs=[
            pl.BlockSpec((2, N_ROWS_BLK, d), lambda i: (0, i, 0)),
            pl.BlockSpec((N_ROWS_BLK, d), lambda i: (i, 0)),
            pl.BlockSpec((N_ROWS_BLK, 1), lambda i: (i, 0)),
            pl.BlockSpec((d, d), lambda i: (0, 0)),
            pl.BlockSpec((1, d), lambda i: (0, 0)),
        ],
        out_specs=[
            pl.BlockSpec((N_ROWS_BLK, d), lambda i: (i, 0)),
            pl.BlockSpec((N_ROWS_BLK, d), lambda i: (i, 0)),
        ],
        out_shape=[
            jax.ShapeDtypeStruct((n, d), jnp.float32),
            jax.ShapeDtypeStruct((n, d), jnp.float32),
        ],
    )(S, m1s, dinv2, W2, b2[None, :])


def _k3_body(s_ref, m_ref, dinv_ref, h1_ref, o_ref):
    ssum = s_ref[0] + s_ref[1] + m_ref[...]
    o_ref[...] = jnp.maximum(ssum * dinv_ref[...], 0.0) + h1_ref[...]


def _tc_final_combine(S, m2s, dinv2, h1):
    n, d = m2s.shape
    grid = (n // N_ROWS_BLK,)
    return pl.pallas_call(
        _k3_body,
        grid=grid,
        compiler_params=_TC_PAR,
        in_specs=[
            pl.BlockSpec((2, N_ROWS_BLK, d), lambda i: (0, i, 0)),
            pl.BlockSpec((N_ROWS_BLK, d), lambda i: (i, 0)),
            pl.BlockSpec((N_ROWS_BLK, 1), lambda i: (i, 0)),
            pl.BlockSpec((N_ROWS_BLK, d), lambda i: (i, 0)),
        ],
        out_specs=pl.BlockSpec((N_ROWS_BLK, d), lambda i: (i, 0)),
        out_shape=jax.ShapeDtypeStruct((n, d), jnp.float32),
    )(S, m2s, dinv2, h1)


def _head_body(hr_ref, hc_ref, zl_ref, rel_ref, w1_ref, w1l_ref, b1_ref,
               w2_ref, b2_ref, lanes_ref, wr2d_ref, brc_ref, ones_ref, o_ref):
    z = (hr_ref[...] * hc_ref[...]).astype(jnp.bfloat16)
    zl = zl_ref[...]
    a = jnp.dot(z, w1_ref[...], preferred_element_type=jnp.float32)
    a = a + zl * w1l_ref[...] + b1_ref[...]
    a = jnp.maximum(a, 0.0).astype(jnp.bfloat16)
    a = jnp.dot(a, w2_ref[...], preferred_element_type=jnp.float32) + b2_ref[...]
    a = jnp.maximum(a, 0.0)
    onehot = jnp.where(rel_ref[...] == lanes_ref[...], 1.0, 0.0)
    w = jnp.dot(onehot, wr2d_ref[...], preferred_element_type=jnp.float32)
    s1 = jnp.dot(a * w, ones_ref[...], preferred_element_type=jnp.float32)
    s2 = jnp.dot(onehot, brc_ref[...], preferred_element_type=jnp.float32)
    o_ref[...] = (s1 + s2)[:, 0]


def _tc_head(hr, hc, zlast2, rel2, Wl1, bl1, Wl2, bl2, Wr, br):
    e, d = hr.shape
    w1m = Wl1[:d].astype(jnp.bfloat16)
    w1l = Wl1[d:d + 1]
    lanes = jnp.arange(4, dtype=jnp.float32)[None, :]
    wr2d = Wr[:, :, 0]
    brc = br
    ones64 = jnp.ones((64, 1), jnp.float32)
    grid = (e // E_BLK,)
    return pl.pallas_call(
        _head_body,
        grid=grid,
        compiler_params=_TC_PAR,
        in_specs=[
            pl.BlockSpec((E_BLK, d), lambda i: (i, 0)),
            pl.BlockSpec((E_BLK, d), lambda i: (i, 0)),
            pl.BlockSpec((E_BLK, 1), lambda i: (i, 0)),
            pl.BlockSpec((E_BLK, 1), lambda i: (i, 0)),
            pl.BlockSpec((d, 256), lambda i: (0, 0)),
            pl.BlockSpec((1, 256), lambda i: (0, 0)),
            pl.BlockSpec((1, 256), lambda i: (0, 0)),
            pl.BlockSpec((256, 64), lambda i: (0, 0)),
            pl.BlockSpec((1, 64), lambda i: (0, 0)),
            pl.BlockSpec((1, 4), lambda i: (0, 0)),
            pl.BlockSpec((4, 64), lambda i: (0, 0)),
            pl.BlockSpec((4, 1), lambda i: (0, 0)),
            pl.BlockSpec((64, 1), lambda i: (0, 0)),
        ],
        out_specs=pl.BlockSpec((E_BLK,), lambda i: (i,)),
        out_shape=jax.ShapeDtypeStruct((e,), jnp.float32),
    )(hr, hc, zlast2, rel2, w1m, w1l, bl1[None, :], Wl2.astype(jnp.bfloat16),
      bl2[None, :], lanes, wr2d, brc, ones64)


_SC_CORES = 2
_SC_TILES = 16
_NW = _SC_CORES * _SC_TILES
_CHUNK = 125
_SEG = 16


def _sc_degree(tei3, zeros_n, n):
    rows_total, chunk = tei3.shape[1:]
    rows_per_tile = rows_total // _NW
    stripe = 624
    last = n - (_SC_TILES - 1) * stripe
    mesh = plsc.VectorSubcoreMesh(core_axis_name="c", subcore_axis_name="s")

    @functools.partial(
        pl.kernel,
        out_type=jax.ShapeDtypeStruct((_SC_CORES * n,), jnp.float32),
        mesh=mesh,
        scratch_types=[
            pltpu.VMEM((rows_per_tile, chunk), jnp.int32),
            pltpu.VMEM((128,), jnp.float32),
            pltpu.VMEM((640,), jnp.float32),
            pltpu.VMEM_SHARED((n,), jnp.float32),
        ],
    )
    def k(tei_hbm, z_hbm, out_hbm, dst_v, ones_v, zbuf, acc):
        del z_hbm
        cid = lax.axis_index("c")
        sid = lax.axis_index("s")
        wid = cid * _SC_TILES + sid
        dst_hbm = tei_hbm.at[1]

        @pl.loop(0, 128, step=16)
        def _(i):
            ones_v[pl.ds(i, 16)] = jnp.full((16,), 1.0, jnp.float32)

        @pl.loop(0, 640, step=16)
        def _(i):
            zbuf[pl.ds(i, 16)] = jnp.zeros((16,), jnp.float32)

        @pl.when(sid < _SC_TILES - 1)
        def _():
            pltpu.sync_copy(zbuf.at[pl.ds(0, stripe)],
                            acc.at[pl.ds(sid * stripe, stripe)])

        @pl.when(sid == _SC_TILES - 1)
        def _():
            pltpu.sync_copy(zbuf.at[pl.ds(0, last)],
                            acc.at[pl.ds(sid * stripe, last)])

        pltpu.sync_copy(dst_hbm.at[pl.ds(wid * rows_per_tile, rows_per_tile)],
                        dst_v)
        plsc.subcore_barrier()

        @pl.loop(0, rows_per_tile)
        def _(r):
            pltpu.sync_copy(ones_v.at[pl.ds(0, chunk)],
                            acc.at[dst_v.at[r]], add=True)

        plsc.subcore_barrier()

        @pl.when(sid < _SC_TILES - 1)
        def _():
            pltpu.sync_copy(acc.at[pl.ds(sid * stripe, stripe)],
                            zbuf.at[pl.ds(0, stripe)])
            pltpu.sync_copy(zbuf.at[pl.ds(0, stripe)],
                            out_hbm.at[pl.ds(cid * n + sid * stripe, stripe)])

        @pl.when(sid == _SC_TILES - 1)
        def _():
            pltpu.sync_copy(acc.at[pl.ds(sid * stripe, last)],
                            zbuf.at[pl.ds(0, last)])
            pltpu.sync_copy(zbuf.at[pl.ds(0, last)],
                            out_hbm.at[pl.ds(cid * n + sid * stripe, last)])

    return k(tei3, zeros_n)


def _sc_segsum(ms, tei3, zeros_nd):
    n, d = ms.shape
    rows_total, chunk = tei3.shape[1:]
    rows_per_tile = rows_total // _NW
    stripe = 624
    last = n - (_SC_TILES - 1) * stripe
    mesh = plsc.VectorSubcoreMesh(core_axis_name="c", subcore_axis_name="s")

    @functools.partial(
        pl.kernel,
        out_type=jax.ShapeDtypeStruct((_SC_CORES, n, d), jnp.float32),
        mesh=mesh,
        scratch_types=[
            pltpu.VMEM((_SEG, chunk), jnp.int32),
            pltpu.VMEM((_SEG, chunk), jnp.int32),
            pltpu.VMEM((chunk, d), jnp.float32),
            pltpu.VMEM((chunk, d), jnp.float32),
            pltpu.VMEM_SHARED((n, d), jnp.float32),
            pltpu.SemaphoreType.DMA,
            pltpu.SemaphoreType.DMA,
        ],
    )
    def k(ms_hbm, tei_hbm, z_hbm, out_hbm,
          src_v, dst_v, buf0, buf1, acc, sem0, sem1):
        cid = lax.axis_index("c")
        sid = lax.axis_index("s")
        wid = cid * _SC_TILES + sid
        src_hbm = tei_hbm.at[0]
        dst_hbm = tei_hbm.at[1]

        @pl.when(sid < _SC_TILES - 1)
        def _():
            pltpu.sync_copy(z_hbm.at[pl.ds(sid * stripe, stripe)],
                            acc.at[pl.ds(sid * stripe, stripe)])

        @pl.when(sid == _SC_TILES - 1)
        def _():
            pltpu.sync_copy(z_hbm.at[pl.ds(sid * stripe, last)],
                            acc.at[pl.ds(sid * stripe, last)])

        plsc.subcore_barrier()

        @pl.loop(0, rows_per_tile, step=_SEG)
        def _(seg):
            base = wid * rows_per_tile + seg
            pltpu.sync_copy(src_hbm.at[pl.ds(base, _SEG)], src_v)
            pltpu.sync_copy(dst_hbm.at[pl.ds(base, _SEG)], dst_v)
            pltpu.async_copy(ms_hbm.at[src_v.at[0]], buf0, sem0)

            @pl.loop(0, _SEG, step=2)
            def _(r):
                pltpu.async_copy(ms_hbm.at[src_v.at[r + 1]], buf1, sem1)
                pltpu.make_async_copy(ms_hbm.at[src_v.at[r]], buf0, sem0).wait()
                pltpu.sync_copy(buf0, acc.at[dst_v.at[r]], add=True)

                @pl.when(r + 2 < _SEG)
                def _():
                    pltpu.async_copy(ms_hbm.at[src_v.at[r + 2]], buf0, sem0)

                pltpu.make_async_copy(ms_hbm.at[src_v.at[r + 1]], buf1, sem1).wait()
                pltpu.sync_copy(buf1, acc.at[dst_v.at[r + 1]], add=True)

        plsc.subcore_barrier()

        @pl.when(sid < _SC_TILES - 1)
        def _():
            pltpu.sync_copy(acc.at[pl.ds(sid * stripe, stripe)],
                            out_hbm.at[cid, pl.ds(sid * stripe, stripe)])

        @pl.when(sid == _SC_TILES - 1)
        def _():
            pltpu.sync_copy(acc.at[pl.ds(sid * stripe, last)],
                            out_hbm.at[cid, pl.ds(sid * stripe, last)])

    return k(ms, tei3, zeros_nd)


def _sc_pair_gather(h, row3d, col3d):
    n, d = h.shape
    nw, rows_per_tile, chunk = row3d.shape
    ep_pad = nw * rows_per_tile * chunk
    mesh = plsc.VectorSubcoreMesh(core_axis_name="c", subcore_axis_name="s")

    @functools.partial(
        pl.kernel,
        out_type=[jax.ShapeDtypeStruct((ep_pad, d), jnp.float32),
                  jax.ShapeDtypeStruct((ep_pad, d), jnp.float32)],
        mesh=mesh,
        scratch_types=[
            pltpu.VMEM((rows_per_tile, chunk), jnp.int32),
            pltpu.VMEM((rows_per_tile, chunk), jnp.int32),
            pltpu.VMEM((chunk, d), jnp.float32),
            pltpu.VMEM((chunk, d), jnp.float32),
            pltpu.SemaphoreType.DMA,
            pltpu.SemaphoreType.DMA,
        ],
    )
    def k(h_hbm, row_hbm, col_hbm, oa_hbm, ob_hbm,
          row_v, col_v, bufa, bufb, sema, semb):
        cid = lax.axis_index("c")
        sid = lax.axis_index("s")
        wid = cid * _SC_TILES + sid
        base = wid * rows_per_tile

        pltpu.sync_copy(row_hbm.at[wid], row_v)
        pltpu.sync_copy(col_hbm.at[wid], col_v)

        @pl.loop(0, rows_per_tile)
        def _(r):
            pltpu.async_copy(h_hbm.at[row_v.at[r]], bufa, sema)
            pltpu.async_copy(h_hbm.at[col_v.at[r]], bufb, semb)
            pltpu.make_async_copy(h_hbm.at[row_v.at[r]], bufa, sema).wait()
            pltpu.make_async_copy(h_hbm.at[col_v.at[r]], bufb, semb).wait()
            pltpu.sync_copy(bufa, oa_hbm.at[pl.ds((base + r) * chunk, chunk)])
            pltpu.sync_copy(bufb, ob_hbm.at[pl.ds((base + r) * chunk, chunk)])

    return k(h, row3d, col3d)


def kernel(x, edge_index, relations, concs, train_edge_index,
           W1, b1, W2, b2, Wl1, bl1, Wl2, bl2, Wr, br):
    n, d = x.shape
    e = train_edge_index.shape[1]
    tei3 = train_edge_index.reshape(2, e // _CHUNK, _CHUNK)
    zeros_n = jnp.zeros((n,), jnp.float32)
    zeros_nd = jnp.zeros((n, d), jnp.float32)

    cnt = _sc_degree(tei3, zeros_n, n)
    dinv2 = jax.lax.rsqrt(1.0 + cnt[:n] + cnt[n:])[:, None]

    m1s = _tc_linear_scale(x, W1, b1[None, :], dinv2)
    S1 = _sc_segsum(m1s, tei3, zeros_nd)
    h1, m2s = _tc_combine_next(S1, m1s, dinv2, W2, b2)
    S2 = _sc_segsum(m2s, tei3, zeros_nd)
    h2 = _tc_final_combine(S2, m2s, dinv2, h1)

    ep = edge_index.shape[0]
    gchunk = 128
    rpt = -(-ep // (_NW * gchunk))
    ep_pad = _NW * rpt * gchunk
    pad = ep_pad - ep
    pad_idx = jnp.arange(pad, dtype=edge_index.dtype)
    row3d = jnp.concatenate(
        [edge_index[:, 0], pad_idx]).reshape(_NW, rpt, gchunk)
    col3d = jnp.concatenate(
        [edge_index[:, 1], pad_idx]).reshape(_NW, rpt, gchunk)
    hr, hc = _sc_pair_gather(h2, row3d, col3d)
    zlast2 = jnp.concatenate(
        [concs[:, 0] * concs[:, 1], jnp.zeros((pad,), jnp.float32)])[:, None]
    rel2 = jnp.concatenate(
        [relations.astype(jnp.float32), jnp.zeros((pad,), jnp.float32)])[:, None]
    out = _tc_head(hr, hc, zlast2, rel2, Wl1, bl1, Wl2, bl2, Wr, br)
    return out[:ep, None]
